# baseline (speedup 1.0000x reference)
import functools

import jax
import jax.numpy as jnp
from jax import lax
from jax.experimental import pallas as pl
from jax.experimental.pallas import tpu as pltpu
from jax.experimental.pallas import tpu_sc as plsc

NC = 2
NS = 16
NW = NC * NS
L = 16
D = 128
DH = D // 2
TE = 256
SB = 128


def _mesh():
    return plsc.VectorSubcoreMesh(core_axis_name="c", subcore_axis_name="s")


@functools.cache
def _edge_att_kernel(N, EP):
    EPW = EP // NW
    TILES = EPW // SB

    @functools.partial(
        pl.kernel,
        out_type=jax.ShapeDtypeStruct((EP,), jnp.float32),
        mesh=_mesh(),
        compiler_params=pltpu.CompilerParams(needs_layout_passes=False),
        scratch_types=[
            pltpu.VMEM((N,), jnp.float32),
            [pltpu.VMEM((1, SB), jnp.int32)] * 2,
            [pltpu.VMEM((1, SB), jnp.int32)] * 2,
            [pltpu.VMEM((SB,), jnp.float32)] * 2,
            [pltpu.VMEM((SB, D), jnp.float32)] * 2,
            [pltpu.VMEM((SB, D), jnp.float32)] * 2,
            pltpu.VMEM((SB,), jnp.float32),
            [pltpu.SemaphoreType.DMA] * 2,
        ],
    )
    def att_kernel(x_hbm, src2_hbm, dst2_hbm, eaf_hbm, n_hbm,
                   att_hbm, n_v, s2b, d2b, eab, xsb, xdb, att_v, sems):
        c = lax.axis_index("c")
        s = lax.axis_index("s")
        base = (s * NC + c) * EPW
        pltpu.sync_copy(n_hbm, n_v)

        def issue(t, b):
            off = pl.multiple_of(base + t * SB, SB)
            row0 = off // SB
            pltpu.sync_copy(src2_hbm.at[pl.ds(row0, 1)], s2b[b])
            pltpu.sync_copy(dst2_hbm.at[pl.ds(row0, 1)], d2b[b])
            pltpu.sync_copy(eaf_hbm.at[pl.ds(off, SB)], eab[b])
            pltpu.async_copy(x_hbm.at[s2b[b].at[0]], xsb[b], sems[b])
            pltpu.async_copy(x_hbm.at[d2b[b].at[0]], xdb[b], sems[b])

        def compute(t, b):
            pltpu.make_async_copy(x_hbm.at[s2b[b].at[0]], xsb[b], sems[b]).wait()
            pltpu.make_async_copy(x_hbm.at[d2b[b].at[0]], xdb[b], sems[b]).wait()
            off = pl.multiple_of(base + t * SB, SB)
            for g in range(SB // L):
                e0 = g * L
                r16 = lax.iota(jnp.int32, L) + e0

                def d_body(d8, acc, r16=r16, b=b):
                    for dd in range(8):
                        colv = jnp.full((L,), d8 * 8 + dd, jnp.int32)
                        a = plsc.load_gather(xsb[b], [r16, colv])
                        bb = plsc.load_gather(xdb[b], [r16, colv])
                        acc = acc + a * bb
                    return acc

                dot16 = lax.fori_loop(0, D // 8, d_body,
                                      jnp.zeros((L,), jnp.float32))
                na = plsc.load_gather(n_v, [s2b[b][0, pl.ds(e0, L)]])
                nb = plsc.load_gather(n_v, [d2b[b][0, pl.ds(e0, L)]])
                ea = eab[b][pl.ds(e0, L)]
                att_v[pl.ds(e0, L)] = dot16 * ea / jnp.maximum(na * nb, 1e-8)
            pltpu.sync_copy(att_v, att_hbm.at[pl.ds(off, SB)])

        issue(0, 0)

        def pipe_body(u, carry):
            t0 = u * 2
            issue(t0 + 1, 1)
            compute(t0, 0)

            @pl.when(t0 + 2 < TILES)
            def _():
                issue(t0 + 2, 0)

            compute(t0 + 1, 1)
            return carry

        lax.fori_loop(0, TILES // 2, pipe_body, 0)

    return att_kernel


@functools.cache
def _aggr_kernel(N, EP):
    EPS = EP // NS
    TILES = EPS // SB
    NR = N // 2
    NPH = 2
    NRA = NR // NPH
    RPT = (NRA // NS) // 8 * 8
    RLAST = NRA - (NS - 1) * RPT

    @functools.partial(
        pl.kernel,
        out_type=jax.ShapeDtypeStruct((2, NR, D), jnp.float32),
        mesh=_mesh(),
        compiler_params=pltpu.CompilerParams(needs_layout_passes=False),
        scratch_types=[
            [pltpu.VMEM((1, SB), jnp.int32)] * 2,
            [pltpu.VMEM((1, SB), jnp.int32)] * 2,
            [pltpu.VMEM((SB,), jnp.int32)] * 2,
            [pltpu.VMEM((SB,), jnp.float32)] * 2,
            [pltpu.VMEM((SB, D), jnp.float32)] * 2,
            [pltpu.VMEM((SB, D), jnp.float32)] * 2,
            pltpu.VMEM_SHARED((NRA, D), jnp.float32),
            [pltpu.SemaphoreType.DMA] * 2,
            [pltpu.SemaphoreType.DMA] * 2,
        ],
    )
    def aggr_kernel(xl_hbm, xlp_hbm, src2_hbm, dstf_hbm, attf_hbm,
                    out_hbm, s2b, sc_v, dfb, attb, xgb, rowsb,
                    aggr_sp, gsems, ssems):
        c = lax.axis_index("c")
        s = lax.axis_index("s")
        coff = c * DH
        r0 = pl.multiple_of(s * RPT, 8)
        stripe = s * EPS

        for p in range(NPH):
            lo = p * NRA

            @pl.when(s < NS - 1)
            def _init_body():
                pltpu.sync_copy(xlp_hbm.at[c, pl.ds(lo + r0, RPT)],
                                aggr_sp.at[pl.ds(r0, RPT)])

            @pl.when(s == NS - 1)
            def _init_tail():
                pltpu.sync_copy(xlp_hbm.at[c, pl.ds(lo + r0, RLAST)],
                                aggr_sp.at[pl.ds(r0, RLAST)])

            plsc.subcore_barrier()

            def issue(t, b):
                off = pl.multiple_of(stripe + t * SB, SB)
                row0 = off // SB
                pltpu.sync_copy(src2_hbm.at[pl.ds(row0, 1)], s2b[b])
                pltpu.sync_copy(dstf_hbm.at[pl.ds(off, SB)], dfb[b])
                pltpu.sync_copy(attf_hbm.at[pl.ds(off, SB)], attb[b])
                pltpu.async_copy(xl_hbm.at[s2b[b].at[0]], xgb[b], gsems[b])

            def compute(t, b, wait_scatter, lo=lo):
                if wait_scatter:
                    pltpu.make_async_copy(
                        rowsb[b], aggr_sp.at[sc_v[b].at[0]], ssems[b]).wait()
                pltpu.make_async_copy(
                    xl_hbm.at[s2b[b].at[0]], xgb[b], gsems[b]).wait()
                for g in range(SB // L):
                    e0 = g * L
                    r16 = lax.iota(jnp.int32, L) + e0
                    d16 = dfb[b][pl.ds(e0, L)]
                    rr = lax.shift_right_logical(d16, 1) - lo
                    inph = (rr >= 0) & (rr < NRA)
                    sc_v[b][0, pl.ds(e0, L)] = jnp.clip(rr, 0, NRA - 1)
                    att16 = jnp.where(inph, attb[b][pl.ds(e0, L)], 0.0)
                    par16 = jnp.bitwise_and(d16, 1).astype(jnp.float32)
                    attp = att16 * par16
                    attq = att16 - attp

                    def d_body(d8, carry2, r16=r16, b=b, attp=attp, attq=attq):
                        for dd in range(8):
                            dcol = d8 * 8 + dd
                            colv = jnp.full((L,), coff + dcol, jnp.int32)
                            v = plsc.load_gather(xgb[b], [r16, colv])
                            cl = jnp.full((L,), dcol, jnp.int32)
                            plsc.store_scatter(rowsb[b], [r16, cl], v * attq)
                            ch = jnp.full((L,), DH + dcol, jnp.int32)
                            plsc.store_scatter(rowsb[b], [r16, ch], v * attp)
                        return carry2

                    lax.fori_loop(0, DH // 8, d_body, 0)
                pltpu.async_copy(rowsb[b], aggr_sp.at[sc_v[b].at[0]], ssems[b],
                                 add=True)

            issue(0, 0)

            def pipe_body(u, carry):
                t0 = u * 2
                issue(t0 + 1, 1)

                @pl.when(u > 0)
                def _():
                    pltpu.make_async_copy(
                        rowsb[0], aggr_sp.at[sc_v[0].at[0]], ssems[0]).wait()

                compute(t0, 0, False)

                @pl.when(t0 + 2 < TILES)
                def _():
                    issue(t0 + 2, 0)

                @pl.when(u > 0)
                def _():
                    pltpu.make_async_copy(
                        rowsb[1], aggr_sp.at[sc_v[1].at[0]], ssems[1]).wait()

                compute(t0 + 1, 1, False)
                return carry

            lax.fori_loop(0, TILES // 2, pipe_body, 0)
            pltpu.make_async_copy(rowsb[0], aggr_sp.at[sc_v[0].at[0]], ssems[0]).wait()
            pltpu.make_async_copy(rowsb[1], aggr_sp.at[sc_v[1].at[0]], ssems[1]).wait()
            plsc.subcore_barrier()

            @pl.when(s < NS - 1)
            def _out_body():
                pltpu.sync_copy(aggr_sp.at[pl.ds(r0, RPT)],
                                out_hbm.at[c, pl.ds(lo + r0, RPT)])

            @pl.when(s == NS - 1)
            def _out_tail():
                pltpu.sync_copy(aggr_sp.at[pl.ds(r0, RLAST)],
                                out_hbm.at[c, pl.ds(lo + r0, RLAST)])

            plsc.subcore_barrier()

    return aggr_kernel


def _pack_paired(xl, BNH):
    z = xl.reshape(BNH, 2, D)
    ev, od = z[:, 0, :], z[:, 1, :]
    return jnp.stack(
        [jnp.concatenate([ev[:, :DH], od[:, :DH]], axis=1),
         jnp.concatenate([ev[:, DH:], od[:, DH:]], axis=1)], axis=0)


def _tc_pre(x, Wt, BN):
    N = x.shape[0]
    BNH = BN // 2

    def body(x_ref, wt_ref, xl_ref, xlp_ref, n_ref):
        xv = x_ref[...]
        xl = jnp.dot(xv, wt_ref[...], preferred_element_type=jnp.float32)
        xl_ref[...] = xl
        xlp_ref[...] = _pack_paired(xl, BNH)
        n_ref[...] = jnp.sqrt(jnp.sum(xv * xv, axis=1, keepdims=True))

    return pl.pallas_call(
        body,
        grid=(N // BN,),
        in_specs=[pl.BlockSpec((BN, D), lambda i: (i, 0)),
                  pl.BlockSpec((D, D), lambda i: (0, 0))],
        out_specs=[pl.BlockSpec((BN, D), lambda i: (i, 0)),
                   pl.BlockSpec((2, BNH, D), lambda i: (0, i, 0)),
                   pl.BlockSpec((BN, 1), lambda i: (i, 0))],
        out_shape=[jax.ShapeDtypeStruct((N, D), jnp.float32),
                   jax.ShapeDtypeStruct((2, N // 2, D), jnp.float32),
                   jax.ShapeDtypeStruct((N, 1), jnp.float32)],
    )(x, Wt)


def _tc_update(aggrp, b2d, Wt, BN):
    NR = aggrp.shape[1]
    N = 2 * NR
    BNH = BN // 2

    def body(a_ref, b_ref, wt_ref, x_ref, xl_ref, xlp_ref, n_ref):
        a = a_ref[...]
        bvec = b_ref[...]
        x0 = jnp.concatenate([a[0, :, :DH], a[1, :, :DH]], axis=1)
        x1 = jnp.concatenate([a[0, :, DH:], a[1, :, DH:]], axis=1)
        h0 = jax.nn.sigmoid(x0 + bvec)
        h1 = jax.nn.sigmoid(x1 + bvec)
        xv = jnp.stack([h0, h1], axis=1).reshape(BN, D)
        x_ref[...] = xv
        xl = jnp.dot(xv, wt_ref[...], preferred_element_type=jnp.float32)
        xl_ref[...] = xl
        xlp_ref[...] = _pack_paired(xl, BNH)
        n_ref[...] = jnp.sqrt(jnp.sum(xv * xv, axis=1, keepdims=True))

    return pl.pallas_call(
        body,
        grid=(N // BN,),
        in_specs=[pl.BlockSpec((2, BNH, D), lambda i: (0, i, 0)),
                  pl.BlockSpec((1, D), lambda i: (0, 0)),
                  pl.BlockSpec((D, D), lambda i: (0, 0))],
        out_specs=[pl.BlockSpec((BN, D), lambda i: (i, 0)),
                   pl.BlockSpec((BN, D), lambda i: (i, 0)),
                   pl.BlockSpec((2, BNH, D), lambda i: (0, i, 0)),
                   pl.BlockSpec((BN, 1), lambda i: (i, 0))],
        out_shape=[jax.ShapeDtypeStruct((N, D), jnp.float32),
                   jax.ShapeDtypeStruct((N, D), jnp.float32),
                   jax.ShapeDtypeStruct((2, NR, D), jnp.float32),
                   jax.ShapeDtypeStruct((N, 1), jnp.float32)],
    )(aggrp, b2d, Wt)


def _pad_edges(edge_index, edge_attr, EP):
    E = edge_index.shape[1]
    zi = jnp.zeros((EP - E,), jnp.int32)
    src = jnp.concatenate([edge_index[0], zi])
    dst = jnp.concatenate([edge_index[1], zi])
    eaf = jnp.concatenate([edge_attr, jnp.zeros((EP - E,), jnp.float32)])
    return (src, dst, eaf,
            src.reshape(EP // SB, SB), dst.reshape(EP // SB, SB))


def _pad_rows(arr, n_to, axis=0):
    pad = n_to - arr.shape[axis]
    if pad == 0:
        return arr
    shp = list(arr.shape)
    shp[axis] = pad
    return jnp.concatenate([arr, jnp.zeros(shp, arr.dtype)], axis=axis)


def kernel(x_ii, edge_index_ii, edge_attr_ii, x_ui, edge_index_ui, edge_attr_ui,
           W1_ii, b1_ii, W2_ii, b2_ii, W1_ui, b1_ui):
    N_ii = x_ii.shape[0]
    N_ui = x_ui.shape[0]
    E_ui = edge_index_ui.shape[1]
    chunk = NW * TE
    EP = ((E_ui + chunk - 1) // chunk) * chunk
    assert edge_index_ii.shape[1] == E_ui

    sf_ii, df_ii, eaf_ii, s2_ii, d2_ii = _pad_edges(edge_index_ii, edge_attr_ii, EP)
    sf_ui, df_ui, eaf_ui, s2_ui, d2_ui = _pad_edges(edge_index_ui, edge_attr_ui, EP)

    attk = _edge_att_kernel(N_ui, EP)
    aggk = _aggr_kernel(N_ui, EP)

    def cgat(x, xl, xlp, n, s2, d2, sf, df, eaf):
        x = _pad_rows(x, N_ui)
        xl = _pad_rows(xl, N_ui)
        xlp = _pad_rows(xlp, N_ui // 2, axis=1)
        n = _pad_rows(n.reshape(-1), N_ui)
        att = attk(x, s2, d2, eaf, n)
        return aggk(xl, xlp, s2, df, att)

    BN = 2000
    xl, xlp, n = _tc_pre(x_ii, W1_ii.T, BN)
    a = cgat(x_ii, xl, xlp, n, s2_ii, d2_ii, sf_ii, df_ii, eaf_ii)[:, :N_ii // 2]
    h1, xl, xlp, n = _tc_update(a, b1_ii.reshape(1, D), W2_ii.T, BN)
    a = cgat(h1, xl, xlp, n, s2_ii, d2_ii, sf_ii, df_ii, eaf_ii)[:, :N_ii // 2]
    h2, _, _, _ = _tc_update(a, b2_ii.reshape(1, D), W2_ii.T, BN)
    xuc = jnp.concatenate([h2, x_ui[N_ii:]], axis=0)
    xl, xlp, n = _tc_pre(xuc, W1_ui.T, BN)
    a = cgat(xuc, xl, xlp, n, s2_ui, d2_ui, sf_ui, df_ui, eaf_ui)
    h3, xl, xlp, n = _tc_update(a, b1_ui.reshape(1, D), W1_ui.T, BN)
    a = cgat(h3, xl, xlp, n, s2_ui, d2_ui, sf_ui, df_ui, eaf_ui)
    h4, _, _, n = _tc_update(a, b1_ui.reshape(1, D), W1_ui.T, BN)
    ones_p = jnp.ones((EP,), jnp.float32)
    att_f = attk(h4, s2_ui, d2_ui, ones_p, n.reshape(-1))
    return att_f[:E_ui]


# speedup vs baseline: 2.1076x; 2.1076x over previous
import functools

import jax
import jax.numpy as jnp
from jax import lax
from jax.experimental import pallas as pl
from jax.experimental.pallas import tpu as pltpu
from jax.experimental.pallas import tpu_sc as plsc

NC = 2
NS = 16
NW = NC * NS
L = 16
D = 128
DH = D // 2
TE = 256
SB = 128


def _mesh():
    return plsc.VectorSubcoreMesh(core_axis_name="c", subcore_axis_name="s")


@functools.cache
def _edge_att_kernel(N, EP):
    EPW = EP // NW
    TILES = EPW // SB

    @functools.partial(
        pl.kernel,
        out_type=jax.ShapeDtypeStruct((EP,), jnp.float32),
        mesh=_mesh(),
        compiler_params=pltpu.CompilerParams(needs_layout_passes=False),
        scratch_types=[
            pltpu.VMEM((N,), jnp.float32),
            [pltpu.VMEM((1, SB), jnp.int32)] * 2,
            [pltpu.VMEM((1, SB), jnp.int32)] * 2,
            [pltpu.VMEM((SB,), jnp.float32)] * 2,
            [pltpu.VMEM((SB, D), jnp.float32)] * 2,
            [pltpu.VMEM((SB, D), jnp.float32)] * 2,
            pltpu.VMEM((SB,), jnp.float32),
            pltpu.VMEM((SB,), jnp.float32),
            [pltpu.SemaphoreType.DMA] * 2,
        ],
    )
    def att_kernel(x_hbm, src2_hbm, dst2_hbm, eaf_hbm, n_hbm,
                   att_hbm, n_v, s2b, d2b, eab, xsb, xdb, att_v, dot_v, sems):
        c = lax.axis_index("c")
        s = lax.axis_index("s")
        base = (s * NC + c) * EPW
        pltpu.sync_copy(n_hbm, n_v)

        def issue(t, b):
            off = pl.multiple_of(base + t * SB, SB)
            row0 = off // SB
            pltpu.sync_copy(src2_hbm.at[pl.ds(row0, 1)], s2b[b])
            pltpu.sync_copy(dst2_hbm.at[pl.ds(row0, 1)], d2b[b])
            pltpu.sync_copy(eaf_hbm.at[pl.ds(off, SB)], eab[b])
            pltpu.async_copy(x_hbm.at[s2b[b].at[0]], xsb[b], sems[b])
            pltpu.async_copy(x_hbm.at[d2b[b].at[0]], xdb[b], sems[b])

        lane = lax.iota(jnp.int32, L)
        last = lane == (L - 1)

        def compute(t, b):
            pltpu.make_async_copy(x_hbm.at[s2b[b].at[0]], xsb[b], sems[b]).wait()
            pltpu.make_async_copy(x_hbm.at[d2b[b].at[0]], xdb[b], sems[b]).wait()
            off = pl.multiple_of(base + t * SB, SB)

            def g_body(g, carry2, b=b):
                e0 = g * L
                for i in range(L):
                    e = e0 + i
                    m = [xsb[b][e, pl.ds(k * L, L)] * xdb[b][e, pl.ds(k * L, L)]
                         for k in range(D // L)]
                    acc = ((m[0] + m[1]) + (m[2] + m[3])) + (
                        (m[4] + m[5]) + (m[6] + m[7]))
                    cum = plsc.cumsum(acc)
                    plsc.store_scatter(dot_v, [jnp.full((L,), e, jnp.int32)],
                                       cum, mask=last)
                dot16 = dot_v[pl.ds(e0, L)]
                na = plsc.load_gather(n_v, [s2b[b][0, pl.ds(e0, L)]])
                nb = plsc.load_gather(n_v, [d2b[b][0, pl.ds(e0, L)]])
                ea = eab[b][pl.ds(e0, L)]
                att_v[pl.ds(e0, L)] = dot16 * ea / jnp.maximum(na * nb, 1e-8)
                return carry2

            lax.fori_loop(0, SB // L, g_body, 0)
            pltpu.sync_copy(att_v, att_hbm.at[pl.ds(off, SB)])

        issue(0, 0)

        def pipe_body(u, carry):
            t0 = u * 2
            issue(t0 + 1, 1)
            compute(t0, 0)

            @pl.when(t0 + 2 < TILES)
            def _():
                issue(t0 + 2, 0)

            compute(t0 + 1, 1)
            return carry

        lax.fori_loop(0, TILES // 2, pipe_body, 0)

    return att_kernel


@functools.cache
def _aggr_kernel(N, EP):
    EPS = EP // NS
    TILES = EPS // SB
    NR = N // 2
    NPH = 2
    NRA = NR // NPH
    RPT = (NRA // NS) // 8 * 8
    RLAST = NRA - (NS - 1) * RPT

    @functools.partial(
        pl.kernel,
        out_type=jax.ShapeDtypeStruct((2, NR, D), jnp.float32),
        mesh=_mesh(),
        compiler_params=pltpu.CompilerParams(needs_layout_passes=False),
        scratch_types=[
            [pltpu.VMEM((1, SB), jnp.int32)] * 2,
            [pltpu.VMEM((1, SB), jnp.int32)] * 2,
            [pltpu.VMEM((SB,), jnp.int32)] * 2,
            [pltpu.VMEM((SB,), jnp.float32)] * 2,
            [pltpu.VMEM((SB, D), jnp.float32)] * 2,
            [pltpu.VMEM((SB, D), jnp.float32)] * 2,
            pltpu.VMEM((SB,), jnp.float32),
            pltpu.VMEM((SB,), jnp.float32),
            pltpu.VMEM_SHARED((NRA, D), jnp.float32),
            [pltpu.SemaphoreType.DMA] * 2,
            [pltpu.SemaphoreType.DMA] * 2,
        ],
    )
    def aggr_kernel(xl_hbm, xlp_hbm, src2_hbm, dstf_hbm, attf_hbm,
                    out_hbm, s2b, sc_v, dfb, attb, xgb, rowsb,
                    attp_v, attq_v, aggr_sp, gsems, ssems):
        c = lax.axis_index("c")
        s = lax.axis_index("s")
        coff = c * DH
        r0 = pl.multiple_of(s * RPT, 8)
        stripe = s * EPS

        for p in range(NPH):
            lo = p * NRA

            @pl.when(s < NS - 1)
            def _init_body():
                pltpu.sync_copy(xlp_hbm.at[c, pl.ds(lo + r0, RPT)],
                                aggr_sp.at[pl.ds(r0, RPT)])

            @pl.when(s == NS - 1)
            def _init_tail():
                pltpu.sync_copy(xlp_hbm.at[c, pl.ds(lo + r0, RLAST)],
                                aggr_sp.at[pl.ds(r0, RLAST)])

            plsc.subcore_barrier()

            def issue(t, b):
                off = pl.multiple_of(stripe + t * SB, SB)
                row0 = off // SB
                pltpu.sync_copy(src2_hbm.at[pl.ds(row0, 1)], s2b[b])
                pltpu.sync_copy(dstf_hbm.at[pl.ds(off, SB)], dfb[b])
                pltpu.sync_copy(attf_hbm.at[pl.ds(off, SB)], attb[b])
                pltpu.async_copy(xl_hbm.at[s2b[b].at[0]], xgb[b], gsems[b])

            def compute(t, b, wait_scatter, lo=lo):
                if wait_scatter:
                    pltpu.make_async_copy(
                        rowsb[b], aggr_sp.at[sc_v[b].at[0]], ssems[b]).wait()
                pltpu.make_async_copy(
                    xl_hbm.at[s2b[b].at[0]], xgb[b], gsems[b]).wait()

                def prep_body(g, carry2, b=b):
                    e0 = g * L
                    d16 = dfb[b][pl.ds(e0, L)]
                    rr = lax.shift_right_logical(d16, 1) - lo
                    inph = (rr >= 0) & (rr < NRA)
                    sc_v[b][0, pl.ds(e0, L)] = jnp.clip(rr, 0, NRA - 1)
                    att16 = jnp.where(inph, attb[b][pl.ds(e0, L)], 0.0)
                    attp16 = att16 * jnp.bitwise_and(d16, 1).astype(jnp.float32)
                    attp_v[pl.ds(e0, L)] = attp16
                    attq_v[pl.ds(e0, L)] = att16 - attp16
                    return carry2

                lax.fori_loop(0, SB // L, prep_body, 0)

                def g_body(g, carry2, b=b):
                    e0 = g * L
                    for i in range(L):
                        e = e0 + i
                        ev = jnp.full((L,), e, jnp.int32)
                        aq = plsc.load_gather(attq_v, [ev])
                        ap = plsc.load_gather(attp_v, [ev])
                        for k in range(DH // L):
                            v = xgb[b][e, pl.ds(coff + k * L, L)]
                            rowsb[b][e, pl.ds(k * L, L)] = v * aq
                            rowsb[b][e, pl.ds(DH + k * L, L)] = v * ap
                    return carry2

                lax.fori_loop(0, SB // L, g_body, 0)
                pltpu.async_copy(rowsb[b], aggr_sp.at[sc_v[b].at[0]], ssems[b],
                                 add=True)

            issue(0, 0)

            def pipe_body(u, carry):
                t0 = u * 2
                issue(t0 + 1, 1)

                @pl.when(u > 0)
                def _():
                    pltpu.make_async_copy(
                        rowsb[0], aggr_sp.at[sc_v[0].at[0]], ssems[0]).wait()

                compute(t0, 0, False)

                @pl.when(t0 + 2 < TILES)
                def _():
                    issue(t0 + 2, 0)

                @pl.when(u > 0)
                def _():
                    pltpu.make_async_copy(
                        rowsb[1], aggr_sp.at[sc_v[1].at[0]], ssems[1]).wait()

                compute(t0 + 1, 1, False)
                return carry

            lax.fori_loop(0, TILES // 2, pipe_body, 0)
            pltpu.make_async_copy(rowsb[0], aggr_sp.at[sc_v[0].at[0]], ssems[0]).wait()
            pltpu.make_async_copy(rowsb[1], aggr_sp.at[sc_v[1].at[0]], ssems[1]).wait()
            plsc.subcore_barrier()

            @pl.when(s < NS - 1)
            def _out_body():
                pltpu.sync_copy(aggr_sp.at[pl.ds(r0, RPT)],
                                out_hbm.at[c, pl.ds(lo + r0, RPT)])

            @pl.when(s == NS - 1)
            def _out_tail():
                pltpu.sync_copy(aggr_sp.at[pl.ds(r0, RLAST)],
                                out_hbm.at[c, pl.ds(lo + r0, RLAST)])

            plsc.subcore_barrier()

    return aggr_kernel


def _pack_paired(xl, BNH):
    z = xl.reshape(BNH, 2, D)
    ev, od = z[:, 0, :], z[:, 1, :]
    return jnp.stack(
        [jnp.concatenate([ev[:, :DH], od[:, :DH]], axis=1),
         jnp.concatenate([ev[:, DH:], od[:, DH:]], axis=1)], axis=0)


def _tc_pre(x, Wt, BN):
    N = x.shape[0]
    BNH = BN // 2

    def body(x_ref, wt_ref, xl_ref, xlp_ref, n_ref):
        xv = x_ref[...]
        xl = jnp.dot(xv, wt_ref[...], preferred_element_type=jnp.float32)
        xl_ref[...] = xl
        xlp_ref[...] = _pack_paired(xl, BNH)
        n_ref[...] = jnp.sqrt(jnp.sum(xv * xv, axis=1, keepdims=True))

    return pl.pallas_call(
        body,
        grid=(N // BN,),
        in_specs=[pl.BlockSpec((BN, D), lambda i: (i, 0)),
                  pl.BlockSpec((D, D), lambda i: (0, 0))],
        out_specs=[pl.BlockSpec((BN, D), lambda i: (i, 0)),
                   pl.BlockSpec((2, BNH, D), lambda i: (0, i, 0)),
                   pl.BlockSpec((BN, 1), lambda i: (i, 0))],
        out_shape=[jax.ShapeDtypeStruct((N, D), jnp.float32),
                   jax.ShapeDtypeStruct((2, N // 2, D), jnp.float32),
                   jax.ShapeDtypeStruct((N, 1), jnp.float32)],
    )(x, Wt)


def _tc_update(aggrp, b2d, Wt, BN):
    NR = aggrp.shape[1]
    N = 2 * NR
    BNH = BN // 2

    def body(a_ref, b_ref, wt_ref, x_ref, xl_ref, xlp_ref, n_ref):
        a = a_ref[...]
        bvec = b_ref[...]
        x0 = jnp.concatenate([a[0, :, :DH], a[1, :, :DH]], axis=1)
        x1 = jnp.concatenate([a[0, :, DH:], a[1, :, DH:]], axis=1)
        h0 = jax.nn.sigmoid(x0 + bvec)
        h1 = jax.nn.sigmoid(x1 + bvec)
        xv = jnp.stack([h0, h1], axis=1).reshape(BN, D)
        x_ref[...] = xv
        xl = jnp.dot(xv, wt_ref[...], preferred_element_type=jnp.float32)
        xl_ref[...] = xl
        xlp_ref[...] = _pack_paired(xl, BNH)
        n_ref[...] = jnp.sqrt(jnp.sum(xv * xv, axis=1, keepdims=True))

    return pl.pallas_call(
        body,
        grid=(N // BN,),
        in_specs=[pl.BlockSpec((2, BNH, D), lambda i: (0, i, 0)),
                  pl.BlockSpec((1, D), lambda i: (0, 0)),
                  pl.BlockSpec((D, D), lambda i: (0, 0))],
        out_specs=[pl.BlockSpec((BN, D), lambda i: (i, 0)),
                   pl.BlockSpec((BN, D), lambda i: (i, 0)),
                   pl.BlockSpec((2, BNH, D), lambda i: (0, i, 0)),
                   pl.BlockSpec((BN, 1), lambda i: (i, 0))],
        out_shape=[jax.ShapeDtypeStruct((N, D), jnp.float32),
                   jax.ShapeDtypeStruct((N, D), jnp.float32),
                   jax.ShapeDtypeStruct((2, NR, D), jnp.float32),
                   jax.ShapeDtypeStruct((N, 1), jnp.float32)],
    )(aggrp, b2d, Wt)


def _pad_edges(edge_index, edge_attr, EP):
    E = edge_index.shape[1]
    zi = jnp.zeros((EP - E,), jnp.int32)
    src = jnp.concatenate([edge_index[0], zi])
    dst = jnp.concatenate([edge_index[1], zi])
    eaf = jnp.concatenate([edge_attr, jnp.zeros((EP - E,), jnp.float32)])
    return (src, dst, eaf,
            src.reshape(EP // SB, SB), dst.reshape(EP // SB, SB))


def _pad_rows(arr, n_to, axis=0):
    pad = n_to - arr.shape[axis]
    if pad == 0:
        return arr
    shp = list(arr.shape)
    shp[axis] = pad
    return jnp.concatenate([arr, jnp.zeros(shp, arr.dtype)], axis=axis)


def kernel(x_ii, edge_index_ii, edge_attr_ii, x_ui, edge_index_ui, edge_attr_ui,
           W1_ii, b1_ii, W2_ii, b2_ii, W1_ui, b1_ui):
    N_ii = x_ii.shape[0]
    N_ui = x_ui.shape[0]
    E_ui = edge_index_ui.shape[1]
    chunk = NW * TE
    EP = ((E_ui + chunk - 1) // chunk) * chunk
    assert edge_index_ii.shape[1] == E_ui

    sf_ii, df_ii, eaf_ii, s2_ii, d2_ii = _pad_edges(edge_index_ii, edge_attr_ii, EP)
    sf_ui, df_ui, eaf_ui, s2_ui, d2_ui = _pad_edges(edge_index_ui, edge_attr_ui, EP)

    attk = _edge_att_kernel(N_ui, EP)
    aggk = _aggr_kernel(N_ui, EP)

    def cgat(x, xl, xlp, n, s2, d2, sf, df, eaf):
        x = _pad_rows(x, N_ui)
        xl = _pad_rows(xl, N_ui)
        xlp = _pad_rows(xlp, N_ui // 2, axis=1)
        n = _pad_rows(n.reshape(-1), N_ui)
        att = attk(x, s2, d2, eaf, n)
        return aggk(xl, xlp, s2, df, att)

    BN = 2000
    xl, xlp, n = _tc_pre(x_ii, W1_ii.T, BN)
    a = cgat(x_ii, xl, xlp, n, s2_ii, d2_ii, sf_ii, df_ii, eaf_ii)[:, :N_ii // 2]
    h1, xl, xlp, n = _tc_update(a, b1_ii.reshape(1, D), W2_ii.T, BN)
    a = cgat(h1, xl, xlp, n, s2_ii, d2_ii, sf_ii, df_ii, eaf_ii)[:, :N_ii // 2]
    h2, _, _, _ = _tc_update(a, b2_ii.reshape(1, D), W2_ii.T, BN)
    xuc = jnp.concatenate([h2, x_ui[N_ii:]], axis=0)
    xl, xlp, n = _tc_pre(xuc, W1_ui.T, BN)
    a = cgat(xuc, xl, xlp, n, s2_ui, d2_ui, sf_ui, df_ui, eaf_ui)
    h3, xl, xlp, n = _tc_update(a, b1_ui.reshape(1, D), W1_ui.T, BN)
    a = cgat(h3, xl, xlp, n, s2_ui, d2_ui, sf_ui, df_ui, eaf_ui)
    h4, _, _, n = _tc_update(a, b1_ui.reshape(1, D), W1_ui.T, BN)
    ones_p = jnp.ones((EP,), jnp.float32)
    att_f = attk(h4, s2_ui, d2_ui, ones_p, n.reshape(-1))
    return att_f[:E_ui]
